# speedup vs baseline: 1.1850x; 1.1850x over previous
import math

import jax
import jax.numpy as jnp
from jax import lax
from jax.experimental import pallas as pl
from jax.experimental.pallas import tpu as pltpu
from jax.experimental.pallas import tpu_sc as plsc

B = 4096
CD = 128
NPOS = 1024
HID = 3584
NSLOT = 6
BM = 512

NWORK = 32
TOTAL = NSLOT * B
BPW = TOTAL // NWORK
GCH = 128


def _index_body(bt_ref, idx_ref):
    scaled = jnp.clip((bt_ref[...] * 1023.0).astype(jnp.int32), 0, 1023)
    x0 = scaled[0:1]
    y0 = scaled[1:2]
    x1 = scaled[2:3]
    y1 = scaled[3:4]
    w = jnp.clip(x1 - x0, 0, 1023)
    h = jnp.clip(y1 - y0, 0, 1023)
    idx_ref[0:1, :] = x0
    idx_ref[1:2, :] = y0 + NPOS
    idx_ref[2:3, :] = x1
    idx_ref[3:4, :] = y1 + NPOS
    idx_ref[4:5, :] = w + 2 * NPOS
    idx_ref[5:6, :] = h + 3 * NPOS


def _compute_indices(bt):
    return pl.pallas_call(
        _index_body,
        out_shape=jax.ShapeDtypeStruct((NSLOT, B), jnp.int32),
    )(bt)


def _sc_gather_body(table_hbm, idx_hbm, out_hbm, idx_v, rows_v, sem):
    wid = lax.axis_index("s") * 2 + lax.axis_index("c")
    base = wid * BPW
    pltpu.sync_copy(idx_hbm.at[pl.ds(base, BPW)], idx_v)
    copies = []
    for j in range(BPW // GCH):
        copies.append(
            pltpu.async_copy(
                table_hbm.at[idx_v.at[pl.ds(j * GCH, GCH)]],
                rows_v.at[pl.ds(j * GCH, GCH)],
                sem,
            )
        )
    for c in copies:
        c.wait()
    pltpu.sync_copy(rows_v, out_hbm.at[pl.ds(base, BPW)])


def _sc_gather(tables, idx_flat):
    mesh = plsc.VectorSubcoreMesh(core_axis_name="c", subcore_axis_name="s")
    return pl.kernel(
        _sc_gather_body,
        out_type=jax.ShapeDtypeStruct((TOTAL, CD), jnp.float32),
        mesh=mesh,
        scratch_types=[
            pltpu.VMEM((BPW,), jnp.int32),
            pltpu.VMEM((BPW, CD), jnp.float32),
            pltpu.SemaphoreType.DMA,
        ],
    )(tables, idx_flat)


def _proj_body(g_ref, w_ref, b_ref, gam_ref, bet_ref, o_ref):
    emb = jnp.concatenate([g_ref[k] for k in range(NSLOT)], axis=1)
    z = lax.dot_general(
        emb,
        w_ref[...],
        (((1,), (0,)), ((), ())),
        precision=lax.Precision.HIGHEST,
        preferred_element_type=jnp.float32,
    )
    z = z + b_ref[...]
    mu = jnp.mean(z, axis=1, keepdims=True)
    d = z - mu
    var = jnp.mean(d * d, axis=1, keepdims=True)
    zn = d * lax.rsqrt(var + 1e-5) * gam_ref[...] + bet_ref[...]
    o_ref[...] = zn * 0.5 * (1.0 + lax.erf(zn * (1.0 / math.sqrt(2.0))))


def _project(g, proj_W, proj_b, ln_gamma, ln_beta):
    return pl.pallas_call(
        _proj_body,
        grid=(B // BM,),
        in_specs=[
            pl.BlockSpec((NSLOT, BM, CD), lambda i: (0, i, 0)),
            pl.BlockSpec((NSLOT * CD, HID), lambda i: (0, 0)),
            pl.BlockSpec((1, HID), lambda i: (0, 0)),
            pl.BlockSpec((1, HID), lambda i: (0, 0)),
            pl.BlockSpec((1, HID), lambda i: (0, 0)),
        ],
        out_specs=pl.BlockSpec((BM, HID), lambda i: (i, 0)),
        out_shape=jax.ShapeDtypeStruct((B, HID), jnp.float32),
    )(g, proj_W, proj_b, ln_gamma, ln_beta)


def kernel(bbox, x_table, y_table, w_table, h_table, proj_W, proj_b, ln_gamma, ln_beta):
    bt = bbox.T
    idx = _compute_indices(bt).reshape(TOTAL)
    tables = jnp.concatenate([x_table, y_table, w_table, h_table], axis=0)
    g = _sc_gather(tables, idx).reshape(NSLOT, B, CD)
    return _project(
        g,
        proj_W,
        proj_b.reshape(1, HID),
        ln_gamma.reshape(1, HID),
        ln_beta.reshape(1, HID),
    )


# speedup vs baseline: 1.9127x; 1.6141x over previous
import math

import jax
import jax.numpy as jnp
from jax import lax
from jax.experimental import pallas as pl
from jax.experimental.pallas import tpu as pltpu
from jax.experimental.pallas import tpu_sc as plsc

B = 4096
CD = 128
NPOS = 1024
HID = 3584
NSLOT = 6
BM = 512

NWORK = 32
TOTAL = NSLOT * B
BPW = TOTAL // NWORK
GCH = 128


def _index_body(bt_ref, idx_ref):
    scaled = jnp.clip((bt_ref[...] * 1023.0).astype(jnp.int32), 0, 1023)
    x0 = scaled[0:1]
    y0 = scaled[1:2]
    x1 = scaled[2:3]
    y1 = scaled[3:4]
    w = jnp.clip(x1 - x0, 0, 1023)
    h = jnp.clip(y1 - y0, 0, 1023)
    idx_ref[0:1, :] = x0
    idx_ref[1:2, :] = y0 + NPOS
    idx_ref[2:3, :] = x1
    idx_ref[3:4, :] = y1 + NPOS
    idx_ref[4:5, :] = w + 2 * NPOS
    idx_ref[5:6, :] = h + 3 * NPOS


def _compute_indices(bt):
    return pl.pallas_call(
        _index_body,
        out_shape=jax.ShapeDtypeStruct((NSLOT, B), jnp.int32),
    )(bt)


def _sc_gather_body(table_hbm, idx_hbm, out_hbm, idx_v, rows_v, sem):
    wid = lax.axis_index("s") * 2 + lax.axis_index("c")
    base = wid * BPW
    pltpu.sync_copy(idx_hbm.at[pl.ds(base, BPW)], idx_v)
    copies = []
    for j in range(BPW // GCH):
        copies.append(
            pltpu.async_copy(
                table_hbm.at[idx_v.at[pl.ds(j * GCH, GCH)]],
                rows_v.at[pl.ds(j * GCH, GCH)],
                sem,
            )
        )
    for c in copies:
        c.wait()
    pltpu.sync_copy(rows_v, out_hbm.at[pl.ds(base, BPW)])


def _sc_gather(tables, idx_flat):
    mesh = plsc.VectorSubcoreMesh(core_axis_name="c", subcore_axis_name="s")
    return pl.kernel(
        _sc_gather_body,
        out_type=jax.ShapeDtypeStruct((TOTAL, CD), jnp.float32),
        mesh=mesh,
        scratch_types=[
            pltpu.VMEM((BPW,), jnp.int32),
            pltpu.VMEM((BPW, CD), jnp.float32),
            pltpu.SemaphoreType.DMA,
        ],
    )(tables, idx_flat)


def _proj_body(g_ref, w_ref, b_ref, gam_ref, bet_ref, o_ref):
    emb = jnp.concatenate([g_ref[k] for k in range(NSLOT)], axis=1)
    z = lax.dot_general(
        emb.astype(jnp.bfloat16),
        w_ref[...].astype(jnp.bfloat16),
        (((1,), (0,)), ((), ())),
        preferred_element_type=jnp.float32,
    )
    z = z + b_ref[...]
    mu = jnp.mean(z, axis=1, keepdims=True)
    d = z - mu
    var = jnp.mean(d * d, axis=1, keepdims=True)
    zn = d * lax.rsqrt(var + 1e-5) * gam_ref[...] + bet_ref[...]
    o_ref[...] = zn * 0.5 * (1.0 + lax.erf(zn * (1.0 / math.sqrt(2.0))))


def _project(g, proj_W, proj_b, ln_gamma, ln_beta):
    return pl.pallas_call(
        _proj_body,
        grid=(B // BM,),
        in_specs=[
            pl.BlockSpec((NSLOT, BM, CD), lambda i: (0, i, 0)),
            pl.BlockSpec((NSLOT * CD, HID), lambda i: (0, 0)),
            pl.BlockSpec((1, HID), lambda i: (0, 0)),
            pl.BlockSpec((1, HID), lambda i: (0, 0)),
            pl.BlockSpec((1, HID), lambda i: (0, 0)),
        ],
        out_specs=pl.BlockSpec((BM, HID), lambda i: (i, 0)),
        out_shape=jax.ShapeDtypeStruct((B, HID), jnp.float32),
    )(g, proj_W, proj_b, ln_gamma, ln_beta)


def kernel(bbox, x_table, y_table, w_table, h_table, proj_W, proj_b, ln_gamma, ln_beta):
    bt = bbox.T
    idx = _compute_indices(bt).reshape(TOTAL)
    tables = jnp.concatenate([x_table, y_table, w_table, h_table], axis=0)
    g = _sc_gather(tables, idx).reshape(NSLOT, B, CD)
    return _project(
        g,
        proj_W,
        proj_b.reshape(1, HID),
        ln_gamma.reshape(1, HID),
        ln_beta.reshape(1, HID),
    )


# speedup vs baseline: 2.0412x; 1.0672x over previous
import math

import jax
import jax.numpy as jnp
from jax import lax
from jax.experimental import pallas as pl
from jax.experimental.pallas import tpu as pltpu
from jax.experimental.pallas import tpu_sc as plsc

B = 4096
CD = 128
NPOS = 1024
HID = 3584
NSLOT = 6
BM = 512

NWORK = 32
TOTAL = NSLOT * B
BPW = TOTAL // NWORK
GCH = 128
PK = CD // 2


def _index_body(bt_ref, idx_ref):
    scaled = jnp.clip((bt_ref[...] * 1023.0).astype(jnp.int32), 0, 1023)
    x0 = scaled[0:1]
    y0 = scaled[1:2]
    x1 = scaled[2:3]
    y1 = scaled[3:4]
    w = jnp.clip(x1 - x0, 0, 1023)
    h = jnp.clip(y1 - y0, 0, 1023)
    idx_ref[0:1, :] = x0
    idx_ref[1:2, :] = y0 + NPOS
    idx_ref[2:3, :] = x1
    idx_ref[3:4, :] = y1 + NPOS
    idx_ref[4:5, :] = w + 2 * NPOS
    idx_ref[5:6, :] = h + 3 * NPOS


def _compute_indices(bt):
    return pl.pallas_call(
        _index_body,
        out_shape=jax.ShapeDtypeStruct((NSLOT, B), jnp.int32),
    )(bt)


def _sc_gather_body(table_hbm, idx_hbm, out_hbm, idx_v, rows_v, sem):
    wid = lax.axis_index("s") * 2 + lax.axis_index("c")
    base = wid * BPW
    pltpu.sync_copy(idx_hbm.at[pl.ds(base, BPW)], idx_v)
    copies = []
    for j in range(BPW // GCH):
        copies.append(
            pltpu.async_copy(
                table_hbm.at[idx_v.at[pl.ds(j * GCH, GCH)]],
                rows_v.at[pl.ds(j * GCH, GCH)],
                sem,
            )
        )
    for c in copies:
        c.wait()
    pltpu.sync_copy(rows_v, out_hbm.at[pl.ds(base, BPW)])


def _sc_gather(tables, idx_flat):
    mesh = plsc.VectorSubcoreMesh(core_axis_name="c", subcore_axis_name="s")
    return pl.kernel(
        _sc_gather_body,
        out_type=jax.ShapeDtypeStruct((TOTAL, PK), jnp.float32),
        mesh=mesh,
        scratch_types=[
            pltpu.VMEM((BPW,), jnp.int32),
            pltpu.VMEM((BPW, PK), jnp.float32),
            pltpu.SemaphoreType.DMA,
        ],
        compiler_params=pltpu.CompilerParams(use_tc_tiling_on_sc=False),
    )(tables, idx_flat)


def _proj_body(g_ref, wlo_ref, whi_ref, b_ref, gam_ref, bet_ref, o_ref):
    packed = jnp.concatenate([g_ref[k] for k in range(NSLOT)], axis=1)
    u = lax.bitcast_convert_type(packed, jnp.int32)
    lo = lax.bitcast_convert_type(u << 16, jnp.float32).astype(jnp.bfloat16)
    hi = lax.bitcast_convert_type(u & jnp.int32(-65536), jnp.float32).astype(jnp.bfloat16)
    dn = (((1,), (0,)), ((), ()))
    z = lax.dot_general(lo, wlo_ref[...], dn, preferred_element_type=jnp.float32)
    z = z + lax.dot_general(hi, whi_ref[...], dn, preferred_element_type=jnp.float32)
    z = z + b_ref[...]
    mu = jnp.mean(z, axis=1, keepdims=True)
    d = z - mu
    var = jnp.mean(d * d, axis=1, keepdims=True)
    zn = d * lax.rsqrt(var + 1e-5) * gam_ref[...] + bet_ref[...]
    o_ref[...] = zn * 0.5 * (1.0 + lax.erf(zn * (1.0 / math.sqrt(2.0))))


def _project(g, w_lo, w_hi, proj_b, ln_gamma, ln_beta):
    return pl.pallas_call(
        _proj_body,
        grid=(B // BM,),
        in_specs=[
            pl.BlockSpec((NSLOT, BM, PK), lambda i: (0, i, 0)),
            pl.BlockSpec((NSLOT * PK, HID), lambda i: (0, 0)),
            pl.BlockSpec((NSLOT * PK, HID), lambda i: (0, 0)),
            pl.BlockSpec((1, HID), lambda i: (0, 0)),
            pl.BlockSpec((1, HID), lambda i: (0, 0)),
            pl.BlockSpec((1, HID), lambda i: (0, 0)),
        ],
        out_specs=pl.BlockSpec((BM, HID), lambda i: (i, 0)),
        out_shape=jax.ShapeDtypeStruct((B, HID), jnp.float32),
    )(g, w_lo, w_hi, proj_b, ln_gamma, ln_beta)


def kernel(bbox, x_table, y_table, w_table, h_table, proj_W, proj_b, ln_gamma, ln_beta):
    bt = bbox.T
    idx = _compute_indices(bt).reshape(TOTAL)
    tables = jnp.concatenate(
        [x_table, y_table, w_table, h_table], axis=0
    ).astype(jnp.bfloat16)
    packed_tables = lax.bitcast_convert_type(
        tables.reshape(4 * NPOS, PK, 2), jnp.float32
    )
    g = _sc_gather(packed_tables, idx).reshape(NSLOT, B, PK)
    w_pair = proj_W.astype(jnp.bfloat16).reshape(NSLOT * PK, 2, HID)
    return _project(
        g,
        w_pair[:, 0],
        w_pair[:, 1],
        proj_b.reshape(1, HID),
        ln_gamma.reshape(1, HID),
        ln_beta.reshape(1, HID),
    )


# speedup vs baseline: 2.1490x; 1.0528x over previous
import math

import jax
import jax.numpy as jnp
from jax import lax
from jax.experimental import pallas as pl
from jax.experimental.pallas import tpu as pltpu
from jax.experimental.pallas import tpu_sc as plsc

B = 4096
CD = 128
NPOS = 1024
HID = 3584
NSLOT = 6
BM = 512
PK = CD // 2

NWORK = 32
TOTAL = NSLOT * B
RPW = B // NWORK
BPW = TOTAL // NWORK
GCH = 128
LANES = 16


def _sc_fused_body(bbox_hbm, table_hbm, out_hbm, bb_v, idx_v, rows_v, sem):
    wid = lax.axis_index("s") * 2 + lax.axis_index("c")
    rbase = wid * RPW
    pltpu.sync_copy(bbox_hbm.at[pl.ds(rbase, RPW)], bb_v)
    lane = lax.iota(jnp.int32, LANES)
    for blk in range(RPW // LANES):
        rowids = lane + blk * LANES
        coords = []
        for c in range(4):
            v = plsc.load_gather(bb_v, [rowids, jnp.full((LANES,), c, jnp.int32)])
            coords.append(
                jnp.clip((v * 1023.0).astype(jnp.int32), 0, 1023)
            )
        x0, y0, x1, y1 = coords
        w = jnp.clip(x1 - x0, 0, 1023)
        h = jnp.clip(y1 - y0, 0, 1023)
        pos = rowids * NSLOT
        plsc.store_scatter(idx_v, [pos], x0)
        plsc.store_scatter(idx_v, [pos + 1], y0 + NPOS)
        plsc.store_scatter(idx_v, [pos + 2], x1)
        plsc.store_scatter(idx_v, [pos + 3], y1 + NPOS)
        plsc.store_scatter(idx_v, [pos + 4], w + 2 * NPOS)
        plsc.store_scatter(idx_v, [pos + 5], h + 3 * NPOS)
    copies = []
    for j in range(BPW // GCH):
        copies.append(
            pltpu.async_copy(
                table_hbm.at[idx_v.at[pl.ds(j * GCH, GCH)]],
                rows_v.at[pl.ds(j * GCH, GCH)],
                sem,
            )
        )
    for c in copies:
        c.wait()
    pltpu.sync_copy(rows_v, out_hbm.at[pl.ds(wid * BPW, BPW)])


def _sc_fused_gather(bbox, packed_tables):
    mesh = plsc.VectorSubcoreMesh(core_axis_name="c", subcore_axis_name="s")
    return pl.kernel(
        _sc_fused_body,
        out_type=jax.ShapeDtypeStruct((TOTAL, PK), jnp.float32),
        mesh=mesh,
        scratch_types=[
            pltpu.VMEM((RPW, 4), jnp.float32),
            pltpu.VMEM((BPW,), jnp.int32),
            pltpu.VMEM((BPW, PK), jnp.float32),
            pltpu.SemaphoreType.DMA,
        ],
        compiler_params=pltpu.CompilerParams(
            use_tc_tiling_on_sc=False, needs_layout_passes=False
        ),
    )(bbox, packed_tables)


def _proj_body(g_ref, wlo_ref, whi_ref, b_ref, gam_ref, bet_ref, o_ref):
    u = lax.bitcast_convert_type(g_ref[...], jnp.int32)
    lo = lax.bitcast_convert_type(u << 16, jnp.float32).astype(jnp.bfloat16)
    hi = lax.bitcast_convert_type(u & jnp.int32(-65536), jnp.float32).astype(jnp.bfloat16)
    dn = (((1,), (0,)), ((), ()))
    z = lax.dot_general(lo, wlo_ref[...], dn, preferred_element_type=jnp.float32)
    z = z + lax.dot_general(hi, whi_ref[...], dn, preferred_element_type=jnp.float32)
    z = z + b_ref[...]
    mu = jnp.mean(z, axis=1, keepdims=True)
    d = z - mu
    var = jnp.mean(d * d, axis=1, keepdims=True)
    zn = d * lax.rsqrt(var + 1e-5) * gam_ref[...] + bet_ref[...]
    o_ref[...] = zn * 0.5 * (1.0 + lax.erf(zn * (1.0 / math.sqrt(2.0))))


def _project(g, w_lo, w_hi, proj_b, ln_gamma, ln_beta):
    return pl.pallas_call(
        _proj_body,
        grid=(B // BM,),
        in_specs=[
            pl.BlockSpec((BM, NSLOT * PK), lambda i: (i, 0)),
            pl.BlockSpec((NSLOT * PK, HID), lambda i: (0, 0)),
            pl.BlockSpec((NSLOT * PK, HID), lambda i: (0, 0)),
            pl.BlockSpec((1, HID), lambda i: (0, 0)),
            pl.BlockSpec((1, HID), lambda i: (0, 0)),
            pl.BlockSpec((1, HID), lambda i: (0, 0)),
        ],
        out_specs=pl.BlockSpec((BM, HID), lambda i: (i, 0)),
        out_shape=jax.ShapeDtypeStruct((B, HID), jnp.float32),
    )(g, w_lo, w_hi, proj_b, ln_gamma, ln_beta)


def kernel(bbox, x_table, y_table, w_table, h_table, proj_W, proj_b, ln_gamma, ln_beta):
    tables = jnp.concatenate(
        [x_table, y_table, w_table, h_table], axis=0
    ).astype(jnp.bfloat16)
    packed_tables = lax.bitcast_convert_type(
        tables.reshape(4 * NPOS, PK, 2), jnp.float32
    )
    g = _sc_fused_gather(bbox, packed_tables).reshape(B, NSLOT * PK)
    w_pair = proj_W.astype(jnp.bfloat16).reshape(NSLOT * PK, 2, HID)
    return _project(
        g,
        w_pair[:, 0],
        w_pair[:, 1],
        proj_b.reshape(1, HID),
        ln_gamma.reshape(1, HID),
        ln_beta.reshape(1, HID),
    )


# speedup vs baseline: 2.4863x; 1.1570x over previous
import math

import jax
import jax.numpy as jnp
from jax import lax
from jax.experimental import pallas as pl
from jax.experimental.pallas import tpu as pltpu
from jax.experimental.pallas import tpu_sc as plsc

B = 4096
CD = 128
NPOS = 1024
HID = 3584
NSLOT = 6
BM = 512
PK = CD // 2
TROWS = 4 * NPOS

NWORK = 32
NSUB = 16
TOTAL = NSLOT * B
RPW = B // NWORK
BPW = TOTAL // NWORK
GCH = 128
NCH = BPW // GCH
LANES = 16


def _sc_body(bbox_hbm, table_hbm, out_hbm,
             bb_v, idx_v, rows_v, shared_tab, tsem, gsem, osem):
    cid = lax.axis_index("c")
    sid = lax.axis_index("s")
    wid = sid * 2 + cid
    trows = TROWS // NSUB
    tstage = pltpu.async_copy(
        table_hbm.at[pl.ds(sid * trows, trows)],
        shared_tab.at[pl.ds(sid * trows, trows)],
        tsem,
    )
    pltpu.sync_copy(bbox_hbm.at[pl.ds(wid * RPW, RPW)], bb_v)
    lane = lax.iota(jnp.int32, LANES)
    for blk in range(RPW // LANES):
        rowids = lane + blk * LANES
        coords = []
        for c in range(4):
            v = plsc.load_gather(bb_v, [rowids, jnp.full((LANES,), c, jnp.int32)])
            coords.append(jnp.clip((v * 1023.0).astype(jnp.int32), 0, 1023))
        x0, y0, x1, y1 = coords
        w = jnp.clip(x1 - x0, 0, 1023)
        h = jnp.clip(y1 - y0, 0, 1023)
        pos = rowids * NSLOT
        plsc.store_scatter(idx_v, [pos], x0)
        plsc.store_scatter(idx_v, [pos + 1], y0 + NPOS)
        plsc.store_scatter(idx_v, [pos + 2], x1)
        plsc.store_scatter(idx_v, [pos + 3], y1 + NPOS)
        plsc.store_scatter(idx_v, [pos + 4], w + 2 * NPOS)
        plsc.store_scatter(idx_v, [pos + 5], h + 3 * NPOS)
    tstage.wait()
    plsc.subcore_barrier()
    gcopies = []
    for j in range(NCH):
        gcopies.append(
            pltpu.async_copy(
                shared_tab.at[idx_v.at[pl.ds(j * GCH, GCH)]],
                rows_v.at[pl.ds(j * GCH, GCH)],
                gsem,
            )
        )
    ocopies = []
    for j in range(NCH):
        gcopies[j].wait()
        ocopies.append(
            pltpu.async_copy(
                rows_v.at[pl.ds(j * GCH, GCH)],
                out_hbm.at[pl.ds(wid * BPW + j * GCH, GCH)],
                osem,
            )
        )
    for c in ocopies:
        c.wait()


def _sc_gather(bbox, packed_tables):
    mesh = plsc.VectorSubcoreMesh(core_axis_name="c", subcore_axis_name="s")
    return pl.kernel(
        _sc_body,
        out_type=jax.ShapeDtypeStruct((TOTAL, PK), jnp.float32),
        mesh=mesh,
        scratch_types=[
            pltpu.VMEM((RPW, 4), jnp.float32),
            pltpu.VMEM((BPW,), jnp.int32),
            pltpu.VMEM((BPW, PK), jnp.float32),
            pltpu.VMEM_SHARED((TROWS, PK), jnp.float32),
            pltpu.SemaphoreType.DMA,
            pltpu.SemaphoreType.DMA,
            pltpu.SemaphoreType.DMA,
        ],
        compiler_params=pltpu.CompilerParams(
            use_tc_tiling_on_sc=False, needs_layout_passes=False
        ),
    )(bbox, packed_tables)


def _proj_body(g_ref, wlo_ref, whi_ref, b_ref, gam_ref, bet_ref, o_ref):
    u = lax.bitcast_convert_type(g_ref[...], jnp.int32)
    lo = lax.bitcast_convert_type(u << 16, jnp.float32).astype(jnp.bfloat16)
    hi = lax.bitcast_convert_type(u & jnp.int32(-65536), jnp.float32).astype(jnp.bfloat16)
    dn = (((1,), (0,)), ((), ()))
    z = lax.dot_general(lo, wlo_ref[...], dn, preferred_element_type=jnp.float32)
    z = z + lax.dot_general(hi, whi_ref[...], dn, preferred_element_type=jnp.float32)
    z = z + b_ref[...]
    mu = jnp.mean(z, axis=1, keepdims=True)
    d = z - mu
    var = jnp.mean(d * d, axis=1, keepdims=True)
    zn = d * lax.rsqrt(var + 1e-5) * gam_ref[...] + bet_ref[...]
    o_ref[...] = zn * 0.5 * (1.0 + lax.erf(zn * (1.0 / math.sqrt(2.0))))


def _project(g, w_lo, w_hi, proj_b, ln_gamma, ln_beta):
    return pl.pallas_call(
        _proj_body,
        grid=(B // BM,),
        in_specs=[
            pl.BlockSpec((BM, NSLOT * PK), lambda i: (i, 0)),
            pl.BlockSpec((NSLOT * PK, HID), lambda i: (0, 0)),
            pl.BlockSpec((NSLOT * PK, HID), lambda i: (0, 0)),
            pl.BlockSpec((1, HID), lambda i: (0, 0)),
            pl.BlockSpec((1, HID), lambda i: (0, 0)),
            pl.BlockSpec((1, HID), lambda i: (0, 0)),
        ],
        out_specs=pl.BlockSpec((BM, HID), lambda i: (i, 0)),
        out_shape=jax.ShapeDtypeStruct((B, HID), jnp.float32),
    )(g, w_lo, w_hi, proj_b, ln_gamma, ln_beta)


def kernel(bbox, x_table, y_table, w_table, h_table, proj_W, proj_b, ln_gamma, ln_beta):
    tables = jnp.concatenate(
        [x_table, y_table, w_table, h_table], axis=0
    ).astype(jnp.bfloat16)
    packed_tables = lax.bitcast_convert_type(
        tables.reshape(TROWS, PK, 2), jnp.float32
    )
    g = _sc_gather(bbox, packed_tables).reshape(B, NSLOT * PK)
    w_pair = proj_W.astype(jnp.bfloat16).reshape(NSLOT * PK, 2, HID)
    return _project(
        g,
        w_pair[:, 0],
        w_pair[:, 1],
        proj_b.reshape(1, HID),
        ln_gamma.reshape(1, HID),
        ln_beta.reshape(1, HID),
    )
